# baseline (speedup 1.0000x reference)
import jax
import jax.numpy as jnp
from jax import lax
from jax.experimental import pallas as pl
from jax.experimental.pallas import tpu as pltpu

B = 64
N = 1_000_000
TOPK = 5
BN = 16384
GRID = (N + BN - 1) // BN
LAST_VALID = N - (GRID - 1) * BN


def _gather_body(lab_sref, *refs):
    block_refs, (x_ref, eq0_ref) = refs[:B], refs[B:]
    cols = lax.broadcasted_iota(jnp.int32, (1, 1, BN), 2)
    for k in range(B):
        off = lab_sref[k] % BN
        v = block_refs[k][...]
        x = jnp.sum(jnp.where(cols == off, v, 0.0))
        x_ref[k, 0] = x
        eq0_ref[k, 0] = jnp.sum(((v == x) & (cols < off)).astype(jnp.int32))


def _mk_gather_spec(k):
    return pl.BlockSpec((1, 1, BN), lambda i, lab, k=k: (k, 0, lab[k] // BN))


_gather = pl.pallas_call(
    _gather_body,
    grid_spec=pltpu.PrefetchScalarGridSpec(
        num_scalar_prefetch=1,
        grid=(1,),
        in_specs=[_mk_gather_spec(k) for k in range(B)],
        out_specs=[
            pl.BlockSpec(memory_space=pltpu.SMEM),
            pl.BlockSpec(memory_space=pltpu.SMEM),
        ],
    ),
    out_shape=[
        jax.ShapeDtypeStruct((B, 1), jnp.float32),
        jax.ShapeDtypeStruct((B, 1), jnp.int32),
    ],
)


def _count_body(x_ref, lab_ref, eq0_ref, logits_ref, out_ref, acc_ref):
    j = pl.program_id(0)

    @pl.when(j == 0)
    def _():
        acc_ref[...] = eq0_ref[...]

    v = logits_ref[...]
    x = x_ref[...]
    fully_before = (j + 1) * BN <= lab_ref[...]
    beats = (v > x) | ((v == x) & fully_before)

    @pl.when(j < GRID - 1)
    def _():
        acc_ref[...] += jnp.sum(beats.astype(jnp.int32), axis=1)[:, None]

    @pl.when(j == GRID - 1)
    def _():
        valid = lax.broadcasted_iota(jnp.int32, (B, BN), 1) < LAST_VALID
        acc = acc_ref[...] + jnp.sum((beats & valid).astype(jnp.int32), axis=1)[:, None]
        out_ref[0, 0] = jnp.sum((acc < TOPK).astype(jnp.float32)) * (1.0 / B)


_count = pl.pallas_call(
    _count_body,
    grid=(GRID,),
    in_specs=[
        pl.BlockSpec((B, 1), lambda j: (0, 0)),
        pl.BlockSpec((B, 1), lambda j: (0, 0)),
        pl.BlockSpec((B, 1), lambda j: (0, 0)),
        pl.BlockSpec((B, BN), lambda j: (0, j)),
    ],
    out_specs=pl.BlockSpec(memory_space=pltpu.SMEM),
    out_shape=jax.ShapeDtypeStruct((1, 1), jnp.float32),
    scratch_shapes=[pltpu.VMEM((B, 1), jnp.int32)],
)


def kernel(logits, labels):
    x, eq0 = _gather(labels, *([logits[:, None, :]] * B))
    out = _count(x, labels[:, None], eq0, logits)
    return out[0, 0]
